# Initial kernel scaffold; baseline (speedup 1.0000x reference)
import jax
import jax.numpy as jnp
from jax.experimental import pallas as pl


def kernel(x, Wqkv, Wproj, Wg, W1, W2, W3, g1, g2):
    raise NotImplementedError("write your pallas kernel here")


# speedup vs baseline: 2.5176x; 2.5176x over previous
import functools

import jax
import jax.numpy as jnp
import numpy as np
from jax.experimental import pallas as pl
from jax.experimental.pallas import tpu as pltpu

_DIM = 768
_NH = 12
_HD = 64
_NE = 8
_TOPK = 2
_HID = 2048
_EPS = 1e-6


def _qkv_body(x_ref, w_ref, g_ref, cos_ref, sin_ref, o_ref):
    xb = x_ref[...]
    ms = jnp.mean(xb * xb, axis=1, keepdims=True)
    xn = xb * jax.lax.rsqrt(ms + _EPS) * g_ref[...]
    qkv = jnp.dot(xn, w_ref[...], preferred_element_type=jnp.float32)
    cos = cos_ref[...]
    sin = sin_ref[...]
    half = _HD // 2
    for part in range(2):
        for h in range(_NH):
            c0 = part * _DIM + h * _HD
            a = qkv[:, c0:c0 + half]
            b = qkv[:, c0 + half:c0 + _HD]
            o_ref[part * _NH + h, :, :half] = a * cos - b * sin
            o_ref[part * _NH + h, :, half:] = a * sin + b * cos
    for h in range(_NH):
        c0 = 2 * _DIM + h * _HD
        o_ref[2 * _NH + h, :, :] = qkv[:, c0:c0 + _HD]


def _attn_body(q_ref, k_ref, v_ref, o_ref):
    q = q_ref[0]
    k = k_ref[0]
    v = v_ref[0]
    s = jax.lax.dot_general(q, k, (((1,), (1,)), ((), ())),
                            preferred_element_type=jnp.float32)
    s = s * (_HD ** -0.5)
    s = s - jnp.max(s, axis=1, keepdims=True)
    p = jnp.exp(s)
    p = p / jnp.sum(p, axis=1, keepdims=True)
    o_ref[0] = jnp.dot(p, v, preferred_element_type=jnp.float32)


def _router_body(attn_ref, x_ref, wp_ref, g2_ref, wg_ref,
                 x2_ref, xn2_ref, we_ref, aux_ref):
    proj = jnp.dot(attn_ref[0], wp_ref[0], preferred_element_type=jnp.float32)
    for h in range(1, _NH):
        proj = proj + jnp.dot(attn_ref[h], wp_ref[h],
                              preferred_element_type=jnp.float32)
    x2 = x_ref[...] + proj
    x2_ref[...] = x2
    ms = jnp.mean(x2 * x2, axis=1, keepdims=True)
    xn2 = x2 * jax.lax.rsqrt(ms + _EPS) * g2_ref[...]
    xn2_ref[...] = xn2
    gate = jnp.dot(xn2, wg_ref[...], preferred_element_type=jnp.float32)
    lanes = jax.lax.broadcasted_iota(jnp.int32, gate.shape, 1)
    m1 = jnp.max(gate, axis=1, keepdims=True)
    eq1 = gate == m1
    i1 = jnp.min(jnp.where(eq1, lanes, _NE), axis=1, keepdims=True)
    oh1 = (lanes == i1)
    masked = jnp.where(oh1, -jnp.inf, gate)
    m2 = jnp.max(masked, axis=1, keepdims=True)
    eq2 = masked == m2
    i2 = jnp.min(jnp.where(eq2, lanes, _NE), axis=1, keepdims=True)
    oh2 = (lanes == i2)
    e2 = jnp.exp(m2 - m1)
    w1 = 1.0 / (1.0 + e2)
    w2 = e2 * w1
    we_ref[...] = jnp.where(oh1, w1, 0.0) + jnp.where(oh2, w2, 0.0)
    pm = jnp.exp(gate - m1)
    p = pm / jnp.sum(pm, axis=1, keepdims=True)
    usage = jnp.mean(p, axis=0, keepdims=True)
    aux_ref[0, 0] = _NE * jnp.sum(usage * usage)


def _moe_body(xn2_ref, x2_ref, we_ref, w1_ref, w3_ref, w2_ref, o_ref, *,
              lt_size):
    e = pl.program_id(0)
    lt = pl.program_id(1)
    xb = xn2_ref[...]
    sel = jax.lax.broadcasted_iota(jnp.int32, we_ref.shape, 1) == e
    wcol = jnp.sum(jnp.where(sel, we_ref[...], 0.0), axis=1, keepdims=True)
    h1 = jax.lax.dot_general(xb, w1_ref[0], (((1,), (1,)), ((), ())),
                             preferred_element_type=jnp.float32)
    h3 = jax.lax.dot_general(xb, w3_ref[0], (((1,), (1,)), ((), ())),
                             preferred_element_type=jnp.float32)
    h = h1 * jax.lax.logistic(h1) * h3
    y = jax.lax.dot_general(h, w2_ref[0], (((1,), (1,)), ((), ())),
                            preferred_element_type=jnp.float32)
    contrib = wcol * y
    row0 = lt * lt_size

    @pl.when(e == 0)
    def _init():
        o_ref[pl.ds(row0, lt_size), :] = x2_ref[...] + contrib

    @pl.when(e != 0)
    def _acc():
        o_ref[pl.ds(row0, lt_size), :] = (
            o_ref[pl.ds(row0, lt_size), :] + contrib)


def kernel(x, Wqkv, Wproj, Wg, W1, W2, W3, g1, g2):
    Bb, L, D = x.shape
    xf = x.reshape(L, D)

    perm = np.arange(3 * _DIM)
    for part in range(2):
        for h in range(_NH):
            base = part * _DIM + h * _HD
            perm[base:base + _HD] = np.concatenate(
                [np.arange(base, base + _HD, 2),
                 np.arange(base + 1, base + _HD, 2)])
    WqkvT = Wqkv[perm].T

    inv_freq = 1.0 / (10000.0 ** (np.arange(0, _HD, 2, dtype=np.float32)
                                  / _HD))
    t = np.arange(L, dtype=np.float32)
    freqs = np.outer(t, inv_freq)
    cos = jnp.asarray(np.cos(freqs))
    sin = jnp.asarray(np.sin(freqs))

    LT_A = 256
    qkv = pl.pallas_call(
        _qkv_body,
        grid=(L // LT_A,),
        in_specs=[
            pl.BlockSpec((LT_A, D), lambda i: (i, 0)),
            pl.BlockSpec((D, 3 * _DIM), lambda i: (0, 0)),
            pl.BlockSpec((1, D), lambda i: (0, 0)),
            pl.BlockSpec((LT_A, _HD // 2), lambda i: (i, 0)),
            pl.BlockSpec((LT_A, _HD // 2), lambda i: (i, 0)),
        ],
        out_specs=pl.BlockSpec((3 * _NH, LT_A, _HD), lambda i: (0, i, 0)),
        out_shape=jax.ShapeDtypeStruct((3 * _NH, L, _HD), jnp.float32),
    )(xf, WqkvT, g1.reshape(1, D), cos, sin)

    attnc = pl.pallas_call(
        _attn_body,
        grid=(_NH,),
        in_specs=[
            pl.BlockSpec((1, L, _HD), lambda h: (h, 0, 0)),
            pl.BlockSpec((1, L, _HD), lambda h: (h + _NH, 0, 0)),
            pl.BlockSpec((1, L, _HD), lambda h: (h + 2 * _NH, 0, 0)),
        ],
        out_specs=pl.BlockSpec((1, L, _HD), lambda h: (h, 0, 0)),
        out_shape=jax.ShapeDtypeStruct((_NH, L, _HD), jnp.float32),
    )(qkv, qkv, qkv)

    x2, xn2, we, aux = pl.pallas_call(
        _router_body,
        grid=(1,),
        in_specs=[
            pl.BlockSpec((_NH, L, _HD), lambda i: (0, 0, 0)),
            pl.BlockSpec((L, D), lambda i: (0, 0)),
            pl.BlockSpec((_NH, _HD, D), lambda i: (0, 0, 0)),
            pl.BlockSpec((1, D), lambda i: (0, 0)),
            pl.BlockSpec((D, _NE), lambda i: (0, 0)),
        ],
        out_specs=[
            pl.BlockSpec((L, D), lambda i: (0, 0)),
            pl.BlockSpec((L, D), lambda i: (0, 0)),
            pl.BlockSpec((L, _NE), lambda i: (0, 0)),
            pl.BlockSpec(memory_space=pltpu.SMEM),
        ],
        out_shape=[
            jax.ShapeDtypeStruct((L, D), jnp.float32),
            jax.ShapeDtypeStruct((L, D), jnp.float32),
            jax.ShapeDtypeStruct((L, _NE), jnp.float32),
            jax.ShapeDtypeStruct((1, 1), jnp.float32),
        ],
    )(attnc, xf, Wproj.T.reshape(_NH, _HD, D), g2.reshape(1, D), Wg.T)

    LT_D = 512
    y = pl.pallas_call(
        functools.partial(_moe_body, lt_size=LT_D),
        grid=(_NE, L // LT_D),
        in_specs=[
            pl.BlockSpec((LT_D, D), lambda e, lt: (lt, 0)),
            pl.BlockSpec((LT_D, D), lambda e, lt: (lt, 0)),
            pl.BlockSpec((LT_D, _NE), lambda e, lt: (lt, 0)),
            pl.BlockSpec((1, _HID, D), lambda e, lt: (e, 0, 0)),
            pl.BlockSpec((1, _HID, D), lambda e, lt: (e, 0, 0)),
            pl.BlockSpec((1, D, _HID), lambda e, lt: (e, 0, 0)),
        ],
        out_specs=pl.BlockSpec((L, D), lambda e, lt: (0, 0)),
        out_shape=jax.ShapeDtypeStruct((L, D), jnp.float32),
    )(xn2, x2, we, W1, W3, W2)

    return (y.reshape(Bb, L, D), aux[0, 0])
